# Initial kernel scaffold; baseline (speedup 1.0000x reference)
import jax
import jax.numpy as jnp
from jax.experimental import pallas as pl


def kernel(user_x, post_x, ei_rev_engages, ei_followed_by, ei_social, ei_engages, Wl_direct, bl_direct, Wr_direct, Wl_author, bl_author, Wr_author, Wl_social, bl_social, Wr_social, Wl_post, bl_post, Wr_post):
    raise NotImplementedError("write your pallas kernel here")


# speedup vs baseline: 17.8253x; 17.8253x over previous
import functools

import jax
import jax.numpy as jnp
from jax import lax
from jax.experimental import pallas as pl
from jax.experimental.pallas import tpu as pltpu
from jax.experimental.pallas import tpu_sc as plsc

N = 10000
D = 128
H = 64
E = 320000

NSUB = 16
GROUP = 128
GR = E // GROUP
PER_SUB = 157
LAST_START = GR - PER_SUB
OVERLAP = NSUB * PER_SUB - GR
ACC_ROWS = 10240
STRIP = ACC_ROWS // NSUB
CW = 16
SW = 128
NGH = 80
NBUF = 5

W_DIRECT, W_AUTHOR, W_SOCIAL = 1.75, 0.7, 0.3

RB = 2000


def _prep_body(xs_ref, w_ref, out_ref):
    x = xs_ref[0]
    w = w_ref[0]
    out_ref[0] = lax.dot_general(
        x, w, (((1,), (1,)), ((), ())), preferred_element_type=jnp.float32)


def _prep(xs, w_all, xmap):
    nv = w_all.shape[0]
    return pl.pallas_call(
        _prep_body,
        grid=(N // RB, nv),
        in_specs=[
            pl.BlockSpec((1, RB, D), lambda b, v: (xmap(v), b, 0)),
            pl.BlockSpec((1, H, D), lambda b, v: (v, 0, 0)),
        ],
        out_specs=pl.BlockSpec((1, RB, H), lambda b, v: (v, b, 0)),
        out_shape=jax.ShapeDtypeStruct((nv, N, H), jnp.float32),
    )(xs, w_all)


def _agg(z_all, e0, e1, e2, e3):
    mesh = plsc.VectorSubcoreMesh(core_axis_name="c", subcore_axis_name="s")
    out_type = jax.ShapeDtypeStruct((4, ACC_ROWS, SW), jnp.float32)
    scratch = (
        [pltpu.VMEM((NGH, 2, GROUP), jnp.int32)]
        + [pltpu.VMEM((GROUP, H), jnp.float32)] * NBUF
        + [pltpu.VMEM((GROUP, CW), jnp.float32),
           pltpu.VMEM((GROUP, H), jnp.float32),
           pltpu.VMEM((GROUP, CW), jnp.float32),
           pltpu.VMEM((GROUP, CW), jnp.float32),
           pltpu.VMEM_SHARED((ACC_ROWS, H), jnp.float32),
           pltpu.VMEM_SHARED((ACC_ROWS, CW), jnp.float32)]
        + [pltpu.SemaphoreType.DMA] * (3 * NBUF)
    )

    @functools.partial(pl.kernel, out_type=out_type, mesh=mesh,
                       scratch_types=scratch,
                       compiler_params=pltpu.CompilerParams(
                           use_tc_tiling_on_sc=False))
    def k(z_ref, e0r, e1r, e2r, e3r, S_out, stg, *bufs_and_sems):
        rbufs = bufs_and_sems[:NBUF]
        ones_v, zbuf, zcnt, cbuf, acc, acc_cnt = bufs_and_sems[NBUF:NBUF + 6]
        sems = bufs_and_sems[NBUF + 6:]
        sgs = sems[:NBUF]
        sss = sems[NBUF:2 * NBUF]
        scs = sems[2 * NBUF:]
        cid = lax.axis_index("c")
        sid = lax.axis_index("s")

        @pl.loop(0, GROUP)
        def _init(i):
            for c in range(H // 16):
                zbuf[i, pl.ds(c * 16, 16)] = jnp.zeros((16,), jnp.float32)
            zcnt[i, :] = jnp.zeros((16,), jnp.float32)
            ones_v[i, :] = jnp.ones((16,), jnp.float32)

        def process(z, edges, r):
            for kk in range(STRIP // GROUP):
                base = sid * STRIP + kk * GROUP
                pltpu.sync_copy(zbuf, acc.at[pl.ds(base, GROUP)])
                pltpu.sync_copy(zcnt, acc_cnt.at[pl.ds(base, GROUP)])
            plsc.subcore_barrier()

            def gather(t, b):
                pltpu.async_copy(z.at[stg.at[t, 0]], rbufs[b], sgs[b])

            def wait_gather(t, b):
                pltpu.make_async_copy(z.at[stg.at[t, 0]], rbufs[b],
                                      sgs[b]).wait()

            def scatter(t, b):
                pltpu.async_copy(rbufs[b], acc.at[stg.at[t, 1]], sss[b],
                                 add=True)
                pltpu.async_copy(ones_v, acc_cnt.at[stg.at[t, 1]], scs[b],
                                 add=True)

            def wait_scatter(t, b):
                pltpu.make_async_copy(rbufs[b], acc.at[stg.at[t, 1]],
                                      sss[b]).wait()
                pltpu.make_async_copy(ones_v, acc_cnt.at[stg.at[t, 1]],
                                      scs[b]).wait()

            start = jnp.minimum(sid * PER_SUB, LAST_START)
            for h in range(2):
                real = NGH if h == 0 else PER_SUB - NGH
                pltpu.sync_copy(edges.at[pl.ds(start + h * NGH, real)],
                                stg.at[pl.ds(0, real)])
                if h == 1:
                    @pl.loop(real, NGH)
                    def _pad(i):
                        for c in range(GROUP // 16):
                            stg[i, 1, pl.ds(c * 16, 16)] = jnp.full(
                                (16,), N, jnp.int32)
                else:
                    @pl.when(sid == NSUB - 1)
                    def _neut():
                        @pl.loop(0, OVERLAP)
                        def _z(i):
                            for c in range(GROUP // 16):
                                stg[i, 1, pl.ds(c * 16, 16)] = jnp.full(
                                    (16,), N, jnp.int32)

                for t in range(NBUF):
                    gather(t, t)
                    if t >= 2:
                        wait_gather(t - 2, t - 2)
                        scatter(t - 2, t - 2)

                @pl.loop(NBUF, NGH, step=NBUF)
                def _steady(tb):
                    for b in range(NBUF):
                        t = tb + b
                        wait_scatter(t - NBUF, b)
                        gather(t, b)
                        b2 = (b - 2) % NBUF
                        wait_gather(t - 2, b2)
                        scatter(t - 2, b2)

                for g in (NGH - 2, NGH - 1):
                    wait_gather(g, g % NBUF)
                    scatter(g, g % NBUF)
                for g in range(NGH - NBUF, NGH):
                    wait_scatter(g, g % NBUF)
            plsc.subcore_barrier()

            for kk in range(STRIP // GROUP):
                base = sid * STRIP + kk * GROUP
                pltpu.sync_copy(acc.at[pl.ds(base, GROUP)], rbufs[0])
                pltpu.sync_copy(
                    rbufs[0],
                    S_out.at[r].at[pl.ds(base, GROUP), pl.ds(0, H)])
                pltpu.sync_copy(acc_cnt.at[pl.ds(base, GROUP)], cbuf)
                pltpu.sync_copy(
                    cbuf,
                    S_out.at[r].at[pl.ds(base, GROUP), pl.ds(H, CW)])
            plsc.subcore_barrier()

        for slot in range(2):
            @pl.when(cid == 0)
            def _c0():
                process(z_ref.at[slot], (e0r, e1r)[slot], slot)

            @pl.when(cid == 1)
            def _c1():
                process(z_ref.at[2 + slot], (e2r, e3r)[slot], 2 + slot)

    return k(z_all, e0, e1, e2, e3)


RB2 = 1000


def _combine_body(S_ref, ru_ref, rp_ref, bu_ref, bp_ref, u_ref, p_ref):
    def mean(r):
        blk = S_ref[r]
        cnt = jnp.maximum(blk[:, H:H + 1], 1.0)
        return blk[:, 0:H] / cnt

    u = (W_DIRECT * mean(0) + W_AUTHOR * mean(1) + W_SOCIAL * mean(2)
         + ru_ref[...] + bu_ref[0:1, :])
    p = mean(3) + rp_ref[...] + bp_ref[0:1, :]
    u_ref[...] = jnp.maximum(u, 0.0)
    p_ref[...] = jnp.maximum(p, 0.0)


def _combine(S, ru, rp, bu, bp):
    return pl.pallas_call(
        _combine_body,
        grid=(N // RB2,),
        in_specs=[
            pl.BlockSpec((4, RB2, SW), lambda b: (0, b, 0)),
            pl.BlockSpec((RB2, H), lambda b: (b, 0)),
            pl.BlockSpec((RB2, H), lambda b: (b, 0)),
            pl.BlockSpec((8, H), lambda b: (0, 0)),
            pl.BlockSpec((8, H), lambda b: (0, 0)),
        ],
        out_specs=[
            pl.BlockSpec((RB2, H), lambda b: (b, 0)),
            pl.BlockSpec((RB2, H), lambda b: (b, 0)),
        ],
        out_shape=[
            jax.ShapeDtypeStruct((N, H), jnp.float32),
            jax.ShapeDtypeStruct((N, H), jnp.float32),
        ],
    )(S, ru, rp, bu, bp)


def _prep_edges(ei):
    return ei.astype(jnp.int32).reshape(2, GR, GROUP).transpose(1, 0, 2)


def kernel(user_x, post_x, ei_rev_engages, ei_followed_by, ei_social,
           ei_engages, Wl_direct, bl_direct, Wr_direct, Wl_author, bl_author,
           Wr_author, Wl_social, bl_social, Wr_social, Wl_post, bl_post,
           Wr_post):
    Wr_u = W_DIRECT * Wr_direct + W_AUTHOR * Wr_author + W_SOCIAL * Wr_social
    bu = W_DIRECT * bl_direct + W_AUTHOR * bl_author + W_SOCIAL * bl_social

    xs = jnp.stack([post_x, user_x])
    w4 = jnp.stack([Wl_direct, Wl_author, Wl_social, Wl_post])
    z4 = _prep(xs, w4, lambda v: v // 2)
    w2 = jnp.stack([Wr_post, Wr_u])
    zr = _prep(xs, w2, lambda v: v)
    rp, ru = zr[0], zr[1]

    e0 = _prep_edges(ei_rev_engages)
    e1 = _prep_edges(ei_followed_by)
    e2 = _prep_edges(ei_social)
    e3 = _prep_edges(ei_engages)

    S = _agg(z4, e0, e1, e2, e3)

    bu2 = jnp.broadcast_to(bu, (8, H))
    bp2 = jnp.broadcast_to(bl_post, (8, H))
    user_out, post_out = _combine(S, ru, rp, bu2, bp2)
    return (user_out, post_out)


# speedup vs baseline: 18.5820x; 1.0424x over previous
import functools

import jax
import jax.numpy as jnp
from jax import lax
from jax.experimental import pallas as pl
from jax.experimental.pallas import tpu as pltpu
from jax.experimental.pallas import tpu_sc as plsc

N = 10000
D = 128
H = 64
E = 320000

NSUB = 16
GROUP = 128
GR = E // GROUP
PER_SUB = 157
LAST_START = GR - PER_SUB
OVERLAP = NSUB * PER_SUB - GR
ACC_ROWS = 10240
STRIP = ACC_ROWS // NSUB
CW = 16
SW = 128
NGH = 80
NBUF = 5

W_DIRECT, W_AUTHOR, W_SOCIAL = 1.75, 0.7, 0.3

RB = 10000


def _matmul(xs_ref, w_ref):
    return lax.dot_general(
        xs_ref[0], w_ref[0], (((1,), (1,)), ((), ())),
        preferred_element_type=jnp.float32)


def _prep_body(xs_ref, w_ref, out_ref):
    out_ref[0] = _matmul(xs_ref, w_ref)


def _prep_packed_body(xs_ref, w_ref, out_ref):
    y = _matmul(xs_ref, w_ref).reshape(RB // 2, 2, H)
    out_ref[0] = jnp.concatenate([y[:, 0, :], y[:, 1, :]], axis=1)


def _prep(xs, w_all, xmap, packed):
    nv = w_all.shape[0]
    out_shape = ((nv, N // 2, 2 * H) if packed else (nv, N, H))
    blk = ((1, RB // 2, 2 * H) if packed else (1, RB, H))
    return pl.pallas_call(
        _prep_packed_body if packed else _prep_body,
        grid=(N // RB, nv),
        in_specs=[
            pl.BlockSpec((1, RB, D), lambda b, v: (xmap(v), b, 0)),
            pl.BlockSpec((1, H, D), lambda b, v: (v, 0, 0)),
        ],
        out_specs=pl.BlockSpec(blk, lambda b, v: (v, b, 0)),
        out_shape=jax.ShapeDtypeStruct(out_shape, jnp.float32),
    )(xs, w_all)


def _agg(z_all, e0, e1, e2, e3):
    mesh = plsc.VectorSubcoreMesh(core_axis_name="c", subcore_axis_name="s")
    out_type = jax.ShapeDtypeStruct((4, ACC_ROWS, SW), jnp.float32)
    scratch = (
        [pltpu.VMEM((NGH, 2, GROUP), jnp.int32)]
        + [pltpu.VMEM((GROUP, H), jnp.float32)] * NBUF
        + [pltpu.VMEM((GROUP, CW), jnp.float32),
           pltpu.VMEM((GROUP, H), jnp.float32),
           pltpu.VMEM((GROUP, CW), jnp.float32),
           pltpu.VMEM((GROUP, CW), jnp.float32),
           pltpu.VMEM_SHARED((ACC_ROWS, H), jnp.float32),
           pltpu.VMEM_SHARED((ACC_ROWS, CW), jnp.float32)]
        + [pltpu.SemaphoreType.DMA] * (3 * NBUF)
    )

    @functools.partial(pl.kernel, out_type=out_type, mesh=mesh,
                       scratch_types=scratch,
                       compiler_params=pltpu.CompilerParams(
                           use_tc_tiling_on_sc=False))
    def k(z_ref, e0r, e1r, e2r, e3r, S_out, stg, *bufs_and_sems):
        rbufs = bufs_and_sems[:NBUF]
        ones_v, zbuf, zcnt, cbuf, acc, acc_cnt = bufs_and_sems[NBUF:NBUF + 6]
        sems = bufs_and_sems[NBUF + 6:]
        sgs = sems[:NBUF]
        sss = sems[NBUF:2 * NBUF]
        scs = sems[2 * NBUF:]
        cid = lax.axis_index("c")
        sid = lax.axis_index("s")

        @pl.loop(0, GROUP)
        def _init(i):
            for c in range(H // 16):
                zbuf[i, pl.ds(c * 16, 16)] = jnp.zeros((16,), jnp.float32)
            zcnt[i, :] = jnp.zeros((16,), jnp.float32)
            ones_v[i, :] = jnp.ones((16,), jnp.float32)

        def process(z, edges, r):
            for kk in range(STRIP // GROUP):
                base = sid * STRIP + kk * GROUP
                pltpu.sync_copy(zbuf, acc.at[pl.ds(base, GROUP)])
                pltpu.sync_copy(zcnt, acc_cnt.at[pl.ds(base, GROUP)])
            plsc.subcore_barrier()

            def gather(t, b):
                pltpu.async_copy(z.at[stg.at[t, 0]], rbufs[b], sgs[b])

            def wait_gather(t, b):
                pltpu.make_async_copy(z.at[stg.at[t, 0]], rbufs[b],
                                      sgs[b]).wait()

            def scatter(t, b):
                pltpu.async_copy(rbufs[b], acc.at[stg.at[t, 1]], sss[b],
                                 add=True)
                pltpu.async_copy(ones_v, acc_cnt.at[stg.at[t, 1]], scs[b],
                                 add=True)

            def wait_scatter(t, b):
                pltpu.make_async_copy(rbufs[b], acc.at[stg.at[t, 1]],
                                      sss[b]).wait()
                pltpu.make_async_copy(ones_v, acc_cnt.at[stg.at[t, 1]],
                                      scs[b]).wait()

            start = jnp.minimum(sid * PER_SUB, LAST_START)
            for h in range(2):
                real = NGH if h == 0 else PER_SUB - NGH
                pltpu.sync_copy(edges.at[pl.ds(start + h * NGH, real)],
                                stg.at[pl.ds(0, real)])
                if h == 1:
                    @pl.loop(real, NGH)
                    def _pad(i):
                        for c in range(GROUP // 16):
                            stg[i, 1, pl.ds(c * 16, 16)] = jnp.full(
                                (16,), N, jnp.int32)
                else:
                    @pl.when(sid == NSUB - 1)
                    def _neut():
                        @pl.loop(0, OVERLAP)
                        def _z(i):
                            for c in range(GROUP // 16):
                                stg[i, 1, pl.ds(c * 16, 16)] = jnp.full(
                                    (16,), N, jnp.int32)

                for t in range(NBUF):
                    gather(t, t)
                    if t >= 2:
                        wait_gather(t - 2, t - 2)
                        scatter(t - 2, t - 2)

                @pl.loop(NBUF, NGH, step=NBUF)
                def _steady(tb):
                    for b in range(NBUF):
                        t = tb + b
                        wait_scatter(t - NBUF, b)
                        gather(t, b)
                        b2 = (b - 2) % NBUF
                        wait_gather(t - 2, b2)
                        scatter(t - 2, b2)

                for g in (NGH - 2, NGH - 1):
                    wait_gather(g, g % NBUF)
                    scatter(g, g % NBUF)
                for g in range(NGH - NBUF, NGH):
                    wait_scatter(g, g % NBUF)
            plsc.subcore_barrier()

            for kk in range(STRIP // GROUP):
                base = sid * STRIP + kk * GROUP
                pltpu.sync_copy(
                    acc.at[pl.ds(base, GROUP)],
                    S_out.at[r].at[pl.ds(base, GROUP), pl.ds(0, H)])
                pltpu.sync_copy(
                    acc_cnt.at[pl.ds(base, GROUP)],
                    S_out.at[r].at[pl.ds(base, GROUP), pl.ds(H, CW)])
            plsc.subcore_barrier()

        for slot in range(2):
            @pl.when(cid == 0)
            def _c0():
                process(z_ref.at[slot], (e0r, e1r)[slot], slot)

            @pl.when(cid == 1)
            def _c1():
                process(z_ref.at[2 + slot], (e2r, e3r)[slot], 2 + slot)

    return k(z_all, e0, e1, e2, e3)


RB2 = 1000


def _combine_body(S_ref, ru_ref, rp_ref, bu_ref, bp_ref, u_ref, p_ref):
    def mean(r):
        blk = S_ref[r]
        cnt = jnp.maximum(blk[:, H:H + 1], 1.0)
        return blk[:, 0:H] / cnt

    u = (W_DIRECT * mean(0) + W_AUTHOR * mean(1) + W_SOCIAL * mean(2)
         + ru_ref[...] + bu_ref[0:1, :])
    p = mean(3) + rp_ref[...] + bp_ref[0:1, :]
    u_ref[...] = jnp.maximum(u, 0.0)
    p_ref[...] = jnp.maximum(p, 0.0)


def _combine(S, ru, rp, bu, bp):
    return pl.pallas_call(
        _combine_body,
        grid=(N // RB2,),
        in_specs=[
            pl.BlockSpec((4, RB2, SW), lambda b: (0, b, 0)),
            pl.BlockSpec((RB2, H), lambda b: (b, 0)),
            pl.BlockSpec((RB2, H), lambda b: (b, 0)),
            pl.BlockSpec((8, H), lambda b: (0, 0)),
            pl.BlockSpec((8, H), lambda b: (0, 0)),
        ],
        out_specs=[
            pl.BlockSpec((RB2, H), lambda b: (b, 0)),
            pl.BlockSpec((RB2, H), lambda b: (b, 0)),
        ],
        out_shape=[
            jax.ShapeDtypeStruct((N, H), jnp.float32),
            jax.ShapeDtypeStruct((N, H), jnp.float32),
        ],
    )(S, ru, rp, bu, bp)


def _prep_edges(ei):
    return ei.astype(jnp.int32).reshape(2, GR, GROUP).transpose(1, 0, 2)


def kernel(user_x, post_x, ei_rev_engages, ei_followed_by, ei_social,
           ei_engages, Wl_direct, bl_direct, Wr_direct, Wl_author, bl_author,
           Wr_author, Wl_social, bl_social, Wr_social, Wl_post, bl_post,
           Wr_post):
    Wr_u = W_DIRECT * Wr_direct + W_AUTHOR * Wr_author + W_SOCIAL * Wr_social
    bu = W_DIRECT * bl_direct + W_AUTHOR * bl_author + W_SOCIAL * bl_social

    xs = jnp.stack([post_x, user_x])
    w4 = jnp.stack([Wl_direct, Wl_author, Wl_social, Wl_post])
    z4 = _prep(xs, w4, lambda v: v // 2, packed=True).reshape(4, N, H)
    w2 = jnp.stack([Wr_post, Wr_u])
    zr = _prep(xs, w2, lambda v: v, packed=False)
    rp, ru = zr[0], zr[1]

    e0 = _prep_edges(ei_rev_engages)
    e1 = _prep_edges(ei_followed_by)
    e2 = _prep_edges(ei_social)
    e3 = _prep_edges(ei_engages)

    S = _agg(z4, e0, e1, e2, e3)

    bu2 = jnp.broadcast_to(bu, (8, H))
    bp2 = jnp.broadcast_to(bl_post, (8, H))
    user_out, post_out = _combine(S, ru, rp, bu2, bp2)
    return (user_out, post_out)


# speedup vs baseline: 18.7054x; 1.0066x over previous
import functools

import jax
import jax.numpy as jnp
from jax import lax
from jax.experimental import pallas as pl
from jax.experimental.pallas import tpu as pltpu
from jax.experimental.pallas import tpu_sc as plsc

N = 10000
D = 128
H = 64
E = 320000

NSUB = 16
GROUP = 128
GR = E // GROUP
PER_SUB = 157
LAST_START = GR - PER_SUB
OVERLAP = NSUB * PER_SUB - GR
ACC_ROWS = 10240
STRIP = ACC_ROWS // NSUB
CW = 16
SW = 128
NGH = 80
NBUF = 5

W_DIRECT, W_AUTHOR, W_SOCIAL = 1.75, 0.7, 0.3

RB = 10000


def _matmul(xs_ref, w_ref):
    return lax.dot_general(
        xs_ref[0], w_ref[0], (((1,), (1,)), ((), ())),
        preferred_element_type=jnp.float32)


def _prep_body(xs_ref, w_ref, out_ref):
    out_ref[0] = _matmul(xs_ref, w_ref)


def _prep_packed_body(xs_ref, w_ref, out_ref):
    y = _matmul(xs_ref, w_ref).reshape(RB // 2, 2, H)
    out_ref[0] = jnp.concatenate([y[:, 0, :], y[:, 1, :]], axis=1)


def _prep(xs, w_all, xmap, packed):
    nv = w_all.shape[0]
    out_shape = ((nv, N // 2, 2 * H) if packed else (nv, N, H))
    blk = ((1, RB // 2, 2 * H) if packed else (1, RB, H))
    return pl.pallas_call(
        _prep_packed_body if packed else _prep_body,
        grid=(N // RB, nv),
        in_specs=[
            pl.BlockSpec((1, RB, D), lambda b, v: (xmap(v), b, 0)),
            pl.BlockSpec((1, H, D), lambda b, v: (v, 0, 0)),
        ],
        out_specs=pl.BlockSpec(blk, lambda b, v: (v, b, 0)),
        out_shape=jax.ShapeDtypeStruct(out_shape, jnp.float32),
    )(xs, w_all)


def _prep_z(x, w2):
    return pl.pallas_call(
        _prep_packed_body,
        grid=(N // RB, 2),
        in_specs=[
            pl.BlockSpec((1, RB, D), lambda b, v: (0, b, 0)),
            pl.BlockSpec((1, H, D), lambda b, v: (v, 0, 0)),
        ],
        out_specs=pl.BlockSpec((1, RB // 2, 2 * H), lambda b, v: (v, b, 0)),
        out_shape=jax.ShapeDtypeStruct((2, N // 2, 2 * H), jnp.float32),
    )(x[None], w2).reshape(2, N, H)


def _agg(z01, z23, e0, e1, e2, e3):
    mesh = plsc.VectorSubcoreMesh(core_axis_name="c", subcore_axis_name="s")
    out_type = jax.ShapeDtypeStruct((4, ACC_ROWS, SW), jnp.float32)
    scratch = (
        [pltpu.VMEM((NGH, 2, GROUP), jnp.int32)]
        + [pltpu.VMEM((GROUP, H), jnp.float32)] * NBUF
        + [pltpu.VMEM((GROUP, CW), jnp.float32),
           pltpu.VMEM((GROUP, H), jnp.float32),
           pltpu.VMEM((GROUP, CW), jnp.float32),
           pltpu.VMEM((GROUP, CW), jnp.float32),
           pltpu.VMEM_SHARED((ACC_ROWS, H), jnp.float32),
           pltpu.VMEM_SHARED((ACC_ROWS, CW), jnp.float32)]
        + [pltpu.SemaphoreType.DMA] * (3 * NBUF)
    )

    @functools.partial(pl.kernel, out_type=out_type, mesh=mesh,
                       scratch_types=scratch,
                       compiler_params=pltpu.CompilerParams(
                           use_tc_tiling_on_sc=False))
    def k(z01r, z23r, e0r, e1r, e2r, e3r, S_out, stg, *bufs_and_sems):
        rbufs = bufs_and_sems[:NBUF]
        ones_v, zbuf, zcnt, cbuf, acc, acc_cnt = bufs_and_sems[NBUF:NBUF + 6]
        sems = bufs_and_sems[NBUF + 6:]
        sgs = sems[:NBUF]
        sss = sems[NBUF:2 * NBUF]
        scs = sems[2 * NBUF:]
        cid = lax.axis_index("c")
        sid = lax.axis_index("s")

        @pl.loop(0, GROUP)
        def _init(i):
            for c in range(H // 16):
                zbuf[i, pl.ds(c * 16, 16)] = jnp.zeros((16,), jnp.float32)
            zcnt[i, :] = jnp.zeros((16,), jnp.float32)
            ones_v[i, :] = jnp.ones((16,), jnp.float32)

        def process(z, edges, r):
            for kk in range(STRIP // GROUP):
                base = sid * STRIP + kk * GROUP
                pltpu.sync_copy(zbuf, acc.at[pl.ds(base, GROUP)])
                pltpu.sync_copy(zcnt, acc_cnt.at[pl.ds(base, GROUP)])
            plsc.subcore_barrier()

            def gather(t, b):
                pltpu.async_copy(z.at[stg.at[t, 0]], rbufs[b], sgs[b])

            def wait_gather(t, b):
                pltpu.make_async_copy(z.at[stg.at[t, 0]], rbufs[b],
                                      sgs[b]).wait()

            def scatter(t, b):
                pltpu.async_copy(rbufs[b], acc.at[stg.at[t, 1]], sss[b],
                                 add=True)
                pltpu.async_copy(ones_v, acc_cnt.at[stg.at[t, 1]], scs[b],
                                 add=True)

            def wait_scatter(t, b):
                pltpu.make_async_copy(rbufs[b], acc.at[stg.at[t, 1]],
                                      sss[b]).wait()
                pltpu.make_async_copy(ones_v, acc_cnt.at[stg.at[t, 1]],
                                      scs[b]).wait()

            start = jnp.minimum(sid * PER_SUB, LAST_START)
            for h in range(2):
                real = NGH if h == 0 else PER_SUB - NGH
                pltpu.sync_copy(edges.at[pl.ds(start + h * NGH, real)],
                                stg.at[pl.ds(0, real)])
                if h == 1:
                    @pl.loop(real, NGH)
                    def _pad(i):
                        for c in range(GROUP // 16):
                            stg[i, 1, pl.ds(c * 16, 16)] = jnp.full(
                                (16,), N, jnp.int32)
                else:
                    @pl.when(sid == NSUB - 1)
                    def _neut():
                        @pl.loop(0, OVERLAP)
                        def _z(i):
                            for c in range(GROUP // 16):
                                stg[i, 1, pl.ds(c * 16, 16)] = jnp.full(
                                    (16,), N, jnp.int32)

                for t in range(NBUF):
                    gather(t, t)
                    if t >= 2:
                        wait_gather(t - 2, t - 2)
                        scatter(t - 2, t - 2)

                @pl.loop(NBUF, NGH, step=NBUF)
                def _steady(tb):
                    for b in range(NBUF):
                        t = tb + b
                        wait_scatter(t - NBUF, b)
                        gather(t, b)
                        b2 = (b - 2) % NBUF
                        wait_gather(t - 2, b2)
                        scatter(t - 2, b2)

                for g in (NGH - 2, NGH - 1):
                    wait_gather(g, g % NBUF)
                    scatter(g, g % NBUF)
                for g in range(NGH - NBUF, NGH):
                    wait_scatter(g, g % NBUF)
            plsc.subcore_barrier()

            for kk in range(STRIP // GROUP):
                base = sid * STRIP + kk * GROUP
                pltpu.sync_copy(
                    acc.at[pl.ds(base, GROUP)],
                    S_out.at[r].at[pl.ds(base, GROUP), pl.ds(0, H)])
                pltpu.sync_copy(
                    acc_cnt.at[pl.ds(base, GROUP)],
                    S_out.at[r].at[pl.ds(base, GROUP), pl.ds(H, CW)])
            plsc.subcore_barrier()

        for slot in range(2):
            @pl.when(cid == 0)
            def _c0():
                process(z01r.at[slot], (e0r, e1r)[slot], slot)

            @pl.when(cid == 1)
            def _c1():
                process(z23r.at[slot], (e2r, e3r)[slot], 2 + slot)

    return k(z01, z23, e0, e1, e2, e3)


RB2 = 1000


def _combine_body(S_ref, ru_ref, rp_ref, bu_ref, bp_ref, u_ref, p_ref):
    def mean(r):
        blk = S_ref[r]
        cnt = jnp.maximum(blk[:, H:H + 1], 1.0)
        return blk[:, 0:H] / cnt

    u = (W_DIRECT * mean(0) + W_AUTHOR * mean(1) + W_SOCIAL * mean(2)
         + ru_ref[...] + bu_ref[0:1, :])
    p = mean(3) + rp_ref[...] + bp_ref[0:1, :]
    u_ref[...] = jnp.maximum(u, 0.0)
    p_ref[...] = jnp.maximum(p, 0.0)


def _combine(S, ru, rp, bu, bp):
    return pl.pallas_call(
        _combine_body,
        grid=(N // RB2,),
        in_specs=[
            pl.BlockSpec((4, RB2, SW), lambda b: (0, b, 0)),
            pl.BlockSpec((RB2, H), lambda b: (b, 0)),
            pl.BlockSpec((RB2, H), lambda b: (b, 0)),
            pl.BlockSpec((8, H), lambda b: (0, 0)),
            pl.BlockSpec((8, H), lambda b: (0, 0)),
        ],
        out_specs=[
            pl.BlockSpec((RB2, H), lambda b: (b, 0)),
            pl.BlockSpec((RB2, H), lambda b: (b, 0)),
        ],
        out_shape=[
            jax.ShapeDtypeStruct((N, H), jnp.float32),
            jax.ShapeDtypeStruct((N, H), jnp.float32),
        ],
    )(S, ru, rp, bu, bp)


def _prep_edges(ei):
    return ei.astype(jnp.int32).reshape(2, GR, GROUP).transpose(1, 0, 2)


def kernel(user_x, post_x, ei_rev_engages, ei_followed_by, ei_social,
           ei_engages, Wl_direct, bl_direct, Wr_direct, Wl_author, bl_author,
           Wr_author, Wl_social, bl_social, Wr_social, Wl_post, bl_post,
           Wr_post):
    Wr_u = W_DIRECT * Wr_direct + W_AUTHOR * Wr_author + W_SOCIAL * Wr_social
    bu = W_DIRECT * bl_direct + W_AUTHOR * bl_author + W_SOCIAL * bl_social

    z01 = _prep_z(post_x, jnp.stack([Wl_direct, Wl_author]))
    z23 = _prep_z(user_x, jnp.stack([Wl_social, Wl_post]))
    xs = jnp.stack([post_x, user_x])
    w2 = jnp.stack([Wr_post, Wr_u])
    zr = _prep(xs, w2, lambda v: v, packed=False)
    rp, ru = zr[0], zr[1]

    e0 = _prep_edges(ei_rev_engages)
    e1 = _prep_edges(ei_followed_by)
    e2 = _prep_edges(ei_social)
    e3 = _prep_edges(ei_engages)

    S = _agg(z01, z23, e0, e1, e2, e3)

    bu2 = jnp.broadcast_to(bu, (8, H))
    bp2 = jnp.broadcast_to(bl_post, (8, H))
    user_out, post_out = _combine(S, ru, rp, bu2, bp2)
    return (user_out, post_out)


# speedup vs baseline: 19.2654x; 1.0299x over previous
import functools

import jax
import jax.numpy as jnp
from jax import lax
from jax.experimental import pallas as pl
from jax.experimental.pallas import tpu as pltpu
from jax.experimental.pallas import tpu_sc as plsc

N = 10000
D = 128
H = 64
E = 320000

NSUB = 16
GROUP = 128
GR = E // GROUP
PER_SUB = 157
LAST_START = GR - PER_SUB
OVERLAP = NSUB * PER_SUB - GR
ACC_ROWS = 10240
STRIP = ACC_ROWS // NSUB
CW = 16
SW = 128
NGH = 80
NBUF = 5

W_DIRECT, W_AUTHOR, W_SOCIAL = 1.75, 0.7, 0.3

RB = 10000


def _matmul(xs_ref, w_ref):
    return lax.dot_general(
        xs_ref[0], w_ref[0], (((1,), (1,)), ((), ())),
        preferred_element_type=jnp.float32)


def _prep_body(xs_ref, w_ref, out_ref):
    out_ref[0] = _matmul(xs_ref, w_ref)


def _prep_packed_body(xs_ref, w_ref, out_ref):
    y = _matmul(xs_ref, w_ref).reshape(RB // 2, 2, H)
    out_ref[0] = jnp.concatenate([y[:, 0, :], y[:, 1, :]], axis=1)


def _prep(xs, w_all, xmap, packed):
    nv = w_all.shape[0]
    out_shape = ((nv, N // 2, 2 * H) if packed else (nv, N, H))
    blk = ((1, RB // 2, 2 * H) if packed else (1, RB, H))
    return pl.pallas_call(
        _prep_packed_body if packed else _prep_body,
        grid=(N // RB, nv),
        in_specs=[
            pl.BlockSpec((1, RB, D), lambda b, v: (xmap(v), b, 0)),
            pl.BlockSpec((1, H, D), lambda b, v: (v, 0, 0)),
        ],
        out_specs=pl.BlockSpec(blk, lambda b, v: (v, b, 0)),
        out_shape=jax.ShapeDtypeStruct(out_shape, jnp.float32),
    )(xs, w_all)


def _prep_z(x, w2):
    return pl.pallas_call(
        _prep_packed_body,
        grid=(N // RB, 2),
        in_specs=[
            pl.BlockSpec((1, RB, D), lambda b, v: (0, b, 0)),
            pl.BlockSpec((1, H, D), lambda b, v: (v, 0, 0)),
        ],
        out_specs=pl.BlockSpec((1, RB // 2, 2 * H), lambda b, v: (v, b, 0)),
        out_shape=jax.ShapeDtypeStruct((2, N // 2, 2 * H), jnp.float32),
    )(x[None], w2).reshape(2, N, H)


def _agg(z01, z23, e0, e1, e2, e3):
    mesh = plsc.VectorSubcoreMesh(core_axis_name="c", subcore_axis_name="s")
    out_type = jax.ShapeDtypeStruct((4, ACC_ROWS, SW), jnp.float32)
    scratch = (
        [pltpu.VMEM((NGH, 2, GROUP), jnp.int32)]
        + [pltpu.VMEM((GROUP, H), jnp.float32)] * NBUF
        + [pltpu.VMEM((GROUP, CW), jnp.float32),
           pltpu.VMEM((GROUP, H), jnp.float32),
           pltpu.VMEM((GROUP, CW), jnp.float32),
           pltpu.VMEM((GROUP, CW), jnp.float32),
           pltpu.VMEM_SHARED((ACC_ROWS, H), jnp.float32),
           pltpu.VMEM_SHARED((ACC_ROWS, CW), jnp.float32)]
        + [pltpu.SemaphoreType.DMA] * (3 * NBUF)
    )

    @functools.partial(pl.kernel, out_type=out_type, mesh=mesh,
                       scratch_types=scratch,
                       compiler_params=pltpu.CompilerParams(
                           use_tc_tiling_on_sc=False))
    def k(z01r, z23r, e0r, e1r, e2r, e3r, S_out, stg, *bufs_and_sems):
        rbufs = bufs_and_sems[:NBUF]
        ones_v, zbuf, zcnt, cbuf, acc, acc_cnt = bufs_and_sems[NBUF:NBUF + 6]
        sems = bufs_and_sems[NBUF + 6:]
        sgs = sems[:NBUF]
        sss = sems[NBUF:2 * NBUF]
        scs = sems[2 * NBUF:]
        cid = lax.axis_index("c")
        sid = lax.axis_index("s")

        @pl.loop(0, GROUP)
        def _init(i):
            for c in range(H // 16):
                zbuf[i, pl.ds(c * 16, 16)] = jnp.zeros((16,), jnp.float32)
            zcnt[i, :] = jnp.zeros((16,), jnp.float32)
            ones_v[i, :] = jnp.ones((16,), jnp.float32)

        def process(z, edges, r):
            for kk in range(STRIP // GROUP):
                base = sid * STRIP + kk * GROUP
                pltpu.sync_copy(zbuf, acc.at[pl.ds(base, GROUP)])
                pltpu.sync_copy(zcnt, acc_cnt.at[pl.ds(base, GROUP)])
            plsc.subcore_barrier()

            def gather(t, b):
                pltpu.async_copy(z.at[stg.at[t, 0]], rbufs[b], sgs[b])

            def wait_gather(t, b):
                pltpu.make_async_copy(z.at[stg.at[t, 0]], rbufs[b],
                                      sgs[b]).wait()

            def scatter(t, b):
                pltpu.async_copy(rbufs[b], acc.at[stg.at[t, 1]], sss[b],
                                 add=True)
                pltpu.async_copy(ones_v, acc_cnt.at[stg.at[t, 1]], scs[b],
                                 add=True)

            def wait_scatter(t, b):
                pltpu.make_async_copy(rbufs[b], acc.at[stg.at[t, 1]],
                                      sss[b]).wait()
                pltpu.make_async_copy(ones_v, acc_cnt.at[stg.at[t, 1]],
                                      scs[b]).wait()

            start = jnp.minimum(sid * PER_SUB, LAST_START)
            for h in range(2):
                real = NGH if h == 0 else PER_SUB - NGH
                pltpu.sync_copy(edges.at[pl.ds(start + h * NGH, real)],
                                stg.at[pl.ds(0, real)])
                if h == 1:
                    @pl.loop(real, NGH)
                    def _pad(i):
                        for c in range(GROUP // 16):
                            stg[i, 1, pl.ds(c * 16, 16)] = jnp.full(
                                (16,), N, jnp.int32)
                else:
                    @pl.when(sid == NSUB - 1)
                    def _neut():
                        @pl.loop(0, OVERLAP)
                        def _z(i):
                            for c in range(GROUP // 16):
                                stg[i, 1, pl.ds(c * 16, 16)] = jnp.full(
                                    (16,), N, jnp.int32)

                for t in range(NBUF):
                    gather(t, t)
                    if t >= 2:
                        wait_gather(t - 2, t - 2)
                        scatter(t - 2, t - 2)

                @pl.loop(NBUF, NGH, step=NBUF)
                def _steady(tb):
                    for b in range(NBUF):
                        t = tb + b
                        wait_scatter(t - NBUF, b)
                        gather(t, b)
                        b2 = (b - 2) % NBUF
                        wait_gather(t - 2, b2)
                        scatter(t - 2, b2)

                for g in (NGH - 2, NGH - 1):
                    wait_gather(g, g % NBUF)
                    scatter(g, g % NBUF)
                for g in range(NGH - NBUF, NGH):
                    wait_scatter(g, g % NBUF)
            plsc.subcore_barrier()

            for kk in range(STRIP // GROUP):
                base = sid * STRIP + kk * GROUP
                pltpu.sync_copy(
                    acc.at[pl.ds(base, GROUP)],
                    S_out.at[r].at[pl.ds(base, GROUP), pl.ds(0, H)])
                pltpu.sync_copy(
                    acc_cnt.at[pl.ds(base, GROUP)],
                    S_out.at[r].at[pl.ds(base, GROUP), pl.ds(H, CW)])
            plsc.subcore_barrier()

        for slot in range(2):
            @pl.when(cid == 0)
            def _c0():
                process(z01r.at[slot], (e0r, e1r)[slot], slot)

            @pl.when(cid == 1)
            def _c1():
                process(z23r.at[slot], (e2r, e3r)[slot], 2 + slot)

    return k(z01, z23, e0, e1, e2, e3)


def _combine_body(S_ref, zr_ref, bu_ref, bp_ref, u_ref, p_ref):
    def mean(r):
        cnt = jnp.maximum(S_ref[r, 0:N, H:H + 1], 1.0)
        return S_ref[r, 0:N, 0:H] / cnt

    u = (W_DIRECT * mean(0) + W_AUTHOR * mean(1) + W_SOCIAL * mean(2)
         + zr_ref[1] + bu_ref[0:1, :])
    p = mean(3) + zr_ref[0] + bp_ref[0:1, :]
    u_ref[...] = jnp.maximum(u, 0.0).T
    p_ref[...] = jnp.maximum(p, 0.0).T


def _combine(S, zr, bu, bp):
    u_t, p_t = pl.pallas_call(
        _combine_body,
        out_shape=[
            jax.ShapeDtypeStruct((H, N), jnp.float32),
            jax.ShapeDtypeStruct((H, N), jnp.float32),
        ],
    )(S, zr, bu, bp)
    return u_t.T, p_t.T


def _prep_edges(ei):
    return ei.astype(jnp.int32).reshape(2, GR, GROUP).transpose(1, 0, 2)


def kernel(user_x, post_x, ei_rev_engages, ei_followed_by, ei_social,
           ei_engages, Wl_direct, bl_direct, Wr_direct, Wl_author, bl_author,
           Wr_author, Wl_social, bl_social, Wr_social, Wl_post, bl_post,
           Wr_post):
    Wr_u = W_DIRECT * Wr_direct + W_AUTHOR * Wr_author + W_SOCIAL * Wr_social
    bu = W_DIRECT * bl_direct + W_AUTHOR * bl_author + W_SOCIAL * bl_social

    z01 = _prep_z(post_x, jnp.stack([Wl_direct, Wl_author]))
    z23 = _prep_z(user_x, jnp.stack([Wl_social, Wl_post]))
    xs = jnp.stack([post_x, user_x])
    w2 = jnp.stack([Wr_post, Wr_u])
    zr = _prep(xs, w2, lambda v: v, packed=False)

    e0 = _prep_edges(ei_rev_engages)
    e1 = _prep_edges(ei_followed_by)
    e2 = _prep_edges(ei_social)
    e3 = _prep_edges(ei_engages)

    S = _agg(z01, z23, e0, e1, e2, e3)

    bu2 = jnp.broadcast_to(bu, (8, H))
    bp2 = jnp.broadcast_to(bl_post, (8, H))
    user_out, post_out = _combine(S, zr, bu2, bp2)
    return (user_out, post_out)


# speedup vs baseline: 19.3799x; 1.0059x over previous
import functools

import jax
import jax.numpy as jnp
from jax import lax
from jax.experimental import pallas as pl
from jax.experimental.pallas import tpu as pltpu
from jax.experimental.pallas import tpu_sc as plsc

N = 10000
D = 128
H = 64
E = 320000

NSUB = 16
GROUP = 128
GR = E // GROUP
PER_SUB = 157
LAST_START = GR - PER_SUB
OVERLAP = NSUB * PER_SUB - GR
ACC_ROWS = 10240
STRIP = ACC_ROWS // NSUB
CW = 16
SW = 128
NGH = 80
NBUF = 5

W_DIRECT, W_AUTHOR, W_SOCIAL = 1.75, 0.7, 0.3

RB = 10000


def _matmul(xs_ref, w_ref):
    return lax.dot_general(
        xs_ref[0], w_ref[0], (((1,), (1,)), ((), ())),
        preferred_element_type=jnp.float32)


def _prep_body(xs_ref, w_ref, out_ref):
    out_ref[0] = _matmul(xs_ref, w_ref)


def _prep_packed_body(xs_ref, w_ref, out_ref):
    y = _matmul(xs_ref, w_ref).reshape(RB // 2, 2, H)
    out_ref[0] = jnp.concatenate([y[:, 0, :], y[:, 1, :]], axis=1)


def _prep(xs, w_all, xmap, packed):
    nv = w_all.shape[0]
    out_shape = ((nv, N // 2, 2 * H) if packed else (nv, N, H))
    blk = ((1, RB // 2, 2 * H) if packed else (1, RB, H))
    return pl.pallas_call(
        _prep_packed_body if packed else _prep_body,
        grid=(N // RB, nv),
        in_specs=[
            pl.BlockSpec((1, RB, D), lambda b, v: (xmap(v), b, 0)),
            pl.BlockSpec((1, H, D), lambda b, v: (v, 0, 0)),
        ],
        out_specs=pl.BlockSpec(blk, lambda b, v: (v, b, 0)),
        out_shape=jax.ShapeDtypeStruct(out_shape, jnp.float32),
    )(xs, w_all)


def _prep_z_body(xp_ref, xu_ref, w_ref, out_ref):
    v = pl.program_id(0)
    x = jnp.where(v == 0, xp_ref[0], xu_ref[0])
    for i in range(2):
        y = lax.dot_general(
            x, w_ref[i], (((1,), (1,)), ((), ())),
            preferred_element_type=jnp.float32).reshape(N // 2, 2, H)
        out_ref[i] = jnp.concatenate([y[:, 0, :], y[:, 1, :]], axis=1)


def _prep_z(post_x, user_x, w4):
    return pl.pallas_call(
        _prep_z_body,
        grid=(2,),
        in_specs=[
            pl.BlockSpec((1, N, D), lambda v: (0, 0, 0)),
            pl.BlockSpec((1, N, D), lambda v: (0, 0, 0)),
            pl.BlockSpec((2, H, D), lambda v: (v, 0, 0)),
        ],
        out_specs=pl.BlockSpec((2, N // 2, 2 * H), lambda v: (v, 0, 0)),
        out_shape=jax.ShapeDtypeStruct((4, N // 2, 2 * H), jnp.float32),
    )(post_x[None], user_x[None], w4).reshape(4, N, H)


def _agg(z4, e0, e1, e2, e3):
    mesh = plsc.VectorSubcoreMesh(core_axis_name="c", subcore_axis_name="s")
    out_type = jax.ShapeDtypeStruct((4, ACC_ROWS, SW), jnp.float32)
    scratch = (
        [pltpu.VMEM((NGH, 2, GROUP), jnp.int32)]
        + [pltpu.VMEM((GROUP, H), jnp.float32)] * NBUF
        + [pltpu.VMEM((GROUP, CW), jnp.float32),
           pltpu.VMEM((GROUP, H), jnp.float32),
           pltpu.VMEM((GROUP, CW), jnp.float32),
           pltpu.VMEM((GROUP, CW), jnp.float32),
           pltpu.VMEM_SHARED((ACC_ROWS, H), jnp.float32),
           pltpu.VMEM_SHARED((ACC_ROWS, CW), jnp.float32)]
        + [pltpu.SemaphoreType.DMA] * (3 * NBUF)
    )

    @functools.partial(pl.kernel, out_type=out_type, mesh=mesh,
                       scratch_types=scratch,
                       compiler_params=pltpu.CompilerParams(
                           use_tc_tiling_on_sc=False))
    def k(z_ref, e0r, e1r, e2r, e3r, S_out, stg, *bufs_and_sems):
        rbufs = bufs_and_sems[:NBUF]
        ones_v, zbuf, zcnt, cbuf, acc, acc_cnt = bufs_and_sems[NBUF:NBUF + 6]
        sems = bufs_and_sems[NBUF + 6:]
        sgs = sems[:NBUF]
        sss = sems[NBUF:2 * NBUF]
        scs = sems[2 * NBUF:]
        cid = lax.axis_index("c")
        sid = lax.axis_index("s")

        @pl.loop(0, GROUP)
        def _init(i):
            for c in range(H // 16):
                zbuf[i, pl.ds(c * 16, 16)] = jnp.zeros((16,), jnp.float32)
            zcnt[i, :] = jnp.zeros((16,), jnp.float32)
            ones_v[i, :] = jnp.ones((16,), jnp.float32)

        def process(z, edges, r):
            for kk in range(STRIP // GROUP):
                base = sid * STRIP + kk * GROUP
                pltpu.sync_copy(zbuf, acc.at[pl.ds(base, GROUP)])
                pltpu.sync_copy(zcnt, acc_cnt.at[pl.ds(base, GROUP)])
            plsc.subcore_barrier()

            def gather(t, b):
                pltpu.async_copy(z.at[stg.at[t, 0]], rbufs[b], sgs[b])

            def wait_gather(t, b):
                pltpu.make_async_copy(z.at[stg.at[t, 0]], rbufs[b],
                                      sgs[b]).wait()

            def scatter(t, b):
                pltpu.async_copy(rbufs[b], acc.at[stg.at[t, 1]], sss[b],
                                 add=True)
                pltpu.async_copy(ones_v, acc_cnt.at[stg.at[t, 1]], scs[b],
                                 add=True)

            def wait_scatter(t, b):
                pltpu.make_async_copy(rbufs[b], acc.at[stg.at[t, 1]],
                                      sss[b]).wait()
                pltpu.make_async_copy(ones_v, acc_cnt.at[stg.at[t, 1]],
                                      scs[b]).wait()

            start = jnp.minimum(sid * PER_SUB, LAST_START)
            for h in range(2):
                real = NGH if h == 0 else PER_SUB - NGH
                pltpu.sync_copy(edges.at[pl.ds(start + h * NGH, real)],
                                stg.at[pl.ds(0, real)])
                if h == 1:
                    @pl.loop(real, NGH)
                    def _pad(i):
                        for c in range(GROUP // 16):
                            stg[i, 1, pl.ds(c * 16, 16)] = jnp.full(
                                (16,), N, jnp.int32)
                else:
                    @pl.when(sid == NSUB - 1)
                    def _neut():
                        @pl.loop(0, OVERLAP)
                        def _z(i):
                            for c in range(GROUP // 16):
                                stg[i, 1, pl.ds(c * 16, 16)] = jnp.full(
                                    (16,), N, jnp.int32)

                for t in range(NBUF):
                    gather(t, t)
                    if t >= 2:
                        wait_gather(t - 2, t - 2)
                        scatter(t - 2, t - 2)

                @pl.loop(NBUF, NGH, step=NBUF)
                def _steady(tb):
                    for b in range(NBUF):
                        t = tb + b
                        wait_scatter(t - NBUF, b)
                        gather(t, b)
                        b2 = (b - 2) % NBUF
                        wait_gather(t - 2, b2)
                        scatter(t - 2, b2)

                for g in (NGH - 2, NGH - 1):
                    wait_gather(g, g % NBUF)
                    scatter(g, g % NBUF)
                for g in range(NGH - NBUF, NGH):
                    wait_scatter(g, g % NBUF)
            plsc.subcore_barrier()

            for kk in range(STRIP // GROUP):
                base = sid * STRIP + kk * GROUP
                pltpu.sync_copy(
                    acc.at[pl.ds(base, GROUP)],
                    S_out.at[r].at[pl.ds(base, GROUP), pl.ds(0, H)])
                pltpu.sync_copy(
                    acc_cnt.at[pl.ds(base, GROUP)],
                    S_out.at[r].at[pl.ds(base, GROUP), pl.ds(H, CW)])
            plsc.subcore_barrier()

        for slot in range(2):
            @pl.when(cid == 0)
            def _c0():
                process(z_ref.at[slot], (e0r, e1r)[slot], slot)

            @pl.when(cid == 1)
            def _c1():
                process(z_ref.at[2 + slot], (e2r, e3r)[slot], 2 + slot)

    return k(z4, e0, e1, e2, e3)


def _combine_body(S_ref, zr_ref, bu_ref, bp_ref, u_ref, p_ref):
    def mean(r):
        cnt = jnp.maximum(S_ref[r, 0:N, H:H + 1], 1.0)
        return S_ref[r, 0:N, 0:H] / cnt

    u = (W_DIRECT * mean(0) + W_AUTHOR * mean(1) + W_SOCIAL * mean(2)
         + zr_ref[1] + bu_ref[0:1, :])
    p = mean(3) + zr_ref[0] + bp_ref[0:1, :]
    u_ref[...] = jnp.maximum(u, 0.0).T
    p_ref[...] = jnp.maximum(p, 0.0).T


def _combine(S, zr, bu, bp):
    u_t, p_t = pl.pallas_call(
        _combine_body,
        out_shape=[
            jax.ShapeDtypeStruct((H, N), jnp.float32),
            jax.ShapeDtypeStruct((H, N), jnp.float32),
        ],
    )(S, zr, bu, bp)
    return u_t.T, p_t.T


def _prep_edges(ei):
    return ei.astype(jnp.int32).reshape(2, GR, GROUP).transpose(1, 0, 2)


def kernel(user_x, post_x, ei_rev_engages, ei_followed_by, ei_social,
           ei_engages, Wl_direct, bl_direct, Wr_direct, Wl_author, bl_author,
           Wr_author, Wl_social, bl_social, Wr_social, Wl_post, bl_post,
           Wr_post):
    Wr_u = W_DIRECT * Wr_direct + W_AUTHOR * Wr_author + W_SOCIAL * Wr_social
    bu = W_DIRECT * bl_direct + W_AUTHOR * bl_author + W_SOCIAL * bl_social

    z4 = _prep_z(post_x, user_x,
                 jnp.stack([Wl_direct, Wl_author, Wl_social, Wl_post]))
    xs = jnp.stack([post_x, user_x])
    w2 = jnp.stack([Wr_post, Wr_u])
    zr = _prep(xs, w2, lambda v: v, packed=False)

    e0 = _prep_edges(ei_rev_engages)
    e1 = _prep_edges(ei_followed_by)
    e2 = _prep_edges(ei_social)
    e3 = _prep_edges(ei_engages)

    S = _agg(z4, e0, e1, e2, e3)

    bu2 = jnp.broadcast_to(bu, (8, H))
    bp2 = jnp.broadcast_to(bl_post, (8, H))
    user_out, post_out = _combine(S, zr, bu2, bp2)
    return (user_out, post_out)
